# baseline (speedup 1.0000x reference)
import functools

import jax
import jax.numpy as jnp
from jax import lax
from jax.experimental import pallas as pl
from jax.experimental.pallas import tpu as pltpu
from jax.experimental.pallas import tpu_sc as plsc

B = 16384
D = 32
NSLOT = 10
NC, NS = 2, 16
NW = NC * NS
BPW = B // NW
CHUNK = 128
NCHUNK = BPW // CHUNK

IDXW = 128
RPACK = 128 // D


def _region_body(table128, idx2d, xr, idx_v, rows, sem):
    w = lax.axis_index("s") * NC + lax.axis_index("c")
    pltpu.sync_copy(idx2d.at[pl.ds(w * NCHUNK, NCHUNK)], idx_v)
    copies = [
        pltpu.async_copy(table128.at[idx_v.at[c]],
                         rows.at[pl.ds(c * CHUNK, CHUNK)], sem)
        for c in range(NCHUNK)
    ]
    for cp in copies:
        cp.wait()
    pltpu.sync_copy(rows, xr.at[pl.ds(w * BPW, BPW)])


def _sc_region(table128, idx2d):
    mesh = plsc.VectorSubcoreMesh(core_axis_name="c", subcore_axis_name="s",
                                  num_cores=NC, num_subcores=NS)
    f = pl.kernel(
        _region_body,
        out_type=jax.ShapeDtypeStruct((B, 128), jnp.float32),
        mesh=mesh,
        scratch_types=[
            pltpu.VMEM((NCHUNK, IDXW), jnp.int32),
            pltpu.VMEM((BPW, 128), jnp.float32),
            pltpu.SemaphoreType.DMA,
        ],
        compiler_params=pltpu.CompilerParams(use_tc_tiling_on_sc=True),
    )
    return f(table128, idx2d)


def _small_body(emb_queue, emb_champ, queue2d, champ2d,
                xq, xc, idx_q, idx_c, rows_q, rows_c, sem):
    w = lax.axis_index("s") * NC + lax.axis_index("c")
    pltpu.sync_copy(queue2d.at[pl.ds(w * NCHUNK, NCHUNK)], idx_q)
    pltpu.sync_copy(champ2d.at[pl.ds(w * NCHUNK * NSLOT, NCHUNK * NSLOT)], idx_c)

    def chunk_step(c, carry):
        base = w * BPW + c * CHUNK
        copies = [pltpu.async_copy(emb_queue.at[idx_q.at[c]], rows_q, sem)]
        for j in range(NSLOT):
            copies.append(
                pltpu.async_copy(emb_champ.at[idx_c.at[c * NSLOT + j]],
                                 rows_c.at[pl.ds(j * CHUNK, CHUNK)], sem))
        for cp in copies:
            cp.wait()
        pltpu.sync_copy(rows_q, xq.at[pl.ds(base, CHUNK)])
        pltpu.sync_copy(rows_c, xc.at[pl.ds(base * NSLOT, CHUNK * NSLOT)])
        return carry

    lax.fori_loop(0, NCHUNK, chunk_step, 0)


def _sc_small(emb_queue, emb_champ, queue2d, champ2d):
    mesh = plsc.VectorSubcoreMesh(core_axis_name="c", subcore_axis_name="s",
                                  num_cores=NC, num_subcores=NS)
    f = pl.kernel(
        _small_body,
        out_type=(
            jax.ShapeDtypeStruct((B, D), jnp.float32),
            jax.ShapeDtypeStruct((B * NSLOT, D), jnp.float32),
        ),
        mesh=mesh,
        scratch_types=[
            pltpu.VMEM((NCHUNK, IDXW), jnp.int32),
            pltpu.VMEM((NCHUNK * NSLOT, IDXW), jnp.int32),
            pltpu.VMEM((CHUNK, D), jnp.float32),
            pltpu.VMEM((CHUNK * NSLOT, D), jnp.float32),
            pltpu.SemaphoreType.DMA,
        ],
        compiler_params=pltpu.CompilerParams(use_tc_tiling_on_sc=False),
    )
    return f(emb_queue, emb_champ, queue2d, champ2d)


MLP_BLK = 2048


def _mlp_body(xrp, rmod, xq, xc, w1r, w1q, w1c, b1, w2, b2, out):
    rm = rmod[...]
    xp = xrp[...]
    xsel = jnp.where(rm == 0, xp[:, 0:D], 0.0)
    for p in range(1, RPACK):
        xsel = xsel + jnp.where(rm == p, xp[:, p * D:(p + 1) * D], 0.0)
    h = (jnp.dot(xsel, w1r[...], preferred_element_type=jnp.float32)
         + jnp.dot(xq[...], w1q[...], preferred_element_type=jnp.float32)
         + jnp.dot(xc[...], w1c[...], preferred_element_type=jnp.float32)
         + b1[...])
    h = jnp.maximum(h, 0.0)
    o = jnp.dot(h, w2[...], preferred_element_type=jnp.float32) + b2[...]
    out[...] = 1.0 / (1.0 + jnp.exp(-o))


def _mlp(xrp, rmod, xq, xc, W1, b1, W2, b2):
    w1r, w1q, w1c = W1[:D], W1[D:2 * D], W1[2 * D:]
    grid = (B // MLP_BLK,)
    return pl.pallas_call(
        _mlp_body,
        grid=grid,
        in_specs=[
            pl.BlockSpec((MLP_BLK, 128), lambda i: (i, 0)),
            pl.BlockSpec((MLP_BLK, 1), lambda i: (i, 0)),
            pl.BlockSpec((MLP_BLK, D), lambda i: (i, 0)),
            pl.BlockSpec((MLP_BLK, NSLOT * D), lambda i: (i, 0)),
            pl.BlockSpec((D, 128), lambda i: (0, 0)),
            pl.BlockSpec((D, 128), lambda i: (0, 0)),
            pl.BlockSpec((NSLOT * D, 128), lambda i: (0, 0)),
            pl.BlockSpec((1, 128), lambda i: (0, 0)),
            pl.BlockSpec((128, 1), lambda i: (0, 0)),
            pl.BlockSpec((1, 1), lambda i: (0, 0)),
        ],
        out_specs=pl.BlockSpec((MLP_BLK, 1), lambda i: (i, 0)),
        out_shape=jax.ShapeDtypeStruct((B, 1), jnp.float32),
    )(xrp, rmod, xq, xc, w1r, w1q, w1c, b1.reshape(1, 128), W2, b2.reshape(1, 1))


def kernel(emb_region, emb_queue, emb_champ, W1, b1, W2, b2, region, queue_type, champion_ids):
    region = region.astype(jnp.int32)
    queue_type = queue_type.astype(jnp.int32)
    champ_flat = champion_ids.astype(jnp.int32).reshape(B * NSLOT)

    table128 = emb_region.reshape(emb_region.shape[0] // RPACK, 128)
    ridx2d = (region // RPACK).reshape(B // IDXW, IDXW)
    rmod = (region % RPACK).reshape(B, 1)
    queue2d = queue_type.reshape(B // IDXW, IDXW)
    champ2d = champ_flat.reshape(B * NSLOT // IDXW, IDXW)

    xrp = _sc_region(table128, ridx2d)
    xq, xc = _sc_small(emb_queue, emb_champ, queue2d, champ2d)
    xc = xc.reshape(B, NSLOT * D)
    out = _mlp(xrp, rmod, xq, xc, W1, b1, W2, b2)
    return jnp.squeeze(out, axis=1)


# speedup vs baseline: 2.7510x; 2.7510x over previous
import functools

import jax
import jax.numpy as jnp
from jax import lax
from jax.experimental import pallas as pl
from jax.experimental.pallas import tpu as pltpu
from jax.experimental.pallas import tpu_sc as plsc

B = 16384
NUM_REGION = 1000000
NUM_QUEUE = 1000
NUM_CHAMP = 200
D = 32
NSLOT = 10
NC, NS = 2, 16
NW = NC * NS
BPW = B // NW
CHUNK = 128
NCHUNK = BPW // CHUNK

IDXW = 128
RPACK = 128 // D


PBG = 8192
PCB = 4 * PBG
PGRID = -(-NUM_REGION // PCB)


def _pack_body(x_ref, o_ref):
    xt = x_ref[...].astype(jnp.bfloat16).T
    parts = [xt[p * PBG:(p + 1) * PBG] for p in range(4)]
    o_ref[...] = jnp.concatenate(parts, axis=1).astype(jnp.float32)


def _pack(table_t):
    return pl.pallas_call(
        _pack_body,
        grid=(PGRID,),
        in_specs=[pl.BlockSpec((D, PCB), lambda i: (0, i))],
        out_specs=pl.BlockSpec((PBG, 128), lambda i: (i, 0)),
        out_shape=jax.ShapeDtypeStruct((PGRID * PBG, 128), jnp.float32),
    )(table_t)


def _region_body(table128, idx2d, xr, idx_v, rows, sem):
    w = lax.axis_index("s") * NC + lax.axis_index("c")
    pltpu.sync_copy(idx2d.at[pl.ds(w * NCHUNK, NCHUNK)], idx_v)
    copies = [
        pltpu.async_copy(table128.at[idx_v.at[c]],
                         rows.at[pl.ds(c * CHUNK, CHUNK)], sem)
        for c in range(NCHUNK)
    ]
    for cp in copies:
        cp.wait()
    pltpu.sync_copy(rows, xr.at[pl.ds(w * BPW, BPW)])


def _sc_region(table128, idx2d):
    mesh = plsc.VectorSubcoreMesh(core_axis_name="c", subcore_axis_name="s",
                                  num_cores=NC, num_subcores=NS)
    f = pl.kernel(
        _region_body,
        out_type=jax.ShapeDtypeStruct((B, 128), jnp.float32),
        mesh=mesh,
        scratch_types=[
            pltpu.VMEM((NCHUNK, IDXW), jnp.int32),
            pltpu.VMEM((BPW, 128), jnp.float32),
            pltpu.SemaphoreType.DMA,
        ],
        compiler_params=pltpu.CompilerParams(use_tc_tiling_on_sc=True),
    )
    return f(table128, idx2d)


def _small_body(pq, queue2d, champ2d,
                hc, idx_q, idx_c, acc0, acc1, sem0, sem1):
    w = lax.axis_index("s") * NC + lax.axis_index("c")
    pltpu.sync_copy(queue2d.at[pl.ds(w * NCHUNK, NCHUNK)], idx_q)
    pltpu.sync_copy(champ2d.at[pl.ds(w * NCHUNK * NSLOT, NCHUNK * NSLOT)], idx_c)
    bufs = [(acc0, sem0), (acc1, sem1)]

    def init(c):
        acc, sem = bufs[c % 2]
        pltpu.async_copy(pq.at[idx_q.at[c]], acc, sem).wait()
        cps = []
        for j in range(NSLOT):
            cps.append(
                pltpu.async_copy(pq.at[idx_c.at[c * NSLOT + j]], acc, sem,
                                 add=True))
        return cps

    def drain_write(c, cps):
        acc, _ = bufs[c % 2]
        for cp in cps:
            cp.wait()
        base = w * BPW + c * CHUNK
        pltpu.sync_copy(acc, hc.at[pl.ds(base, CHUNK)])

    inflight = {0: init(0), 1: init(1)}
    for c in range(NCHUNK):
        drain_write(c, inflight.pop(c))
        nxt = c + 2
        if nxt < NCHUNK:
            inflight[nxt] = init(nxt)


def _sc_small(pq, queue2d, champ2d):
    mesh = plsc.VectorSubcoreMesh(core_axis_name="c", subcore_axis_name="s",
                                  num_cores=NC, num_subcores=NS)
    f = pl.kernel(
        _small_body,
        out_type=jax.ShapeDtypeStruct((B, 128), jnp.float32),
        mesh=mesh,
        scratch_types=[
            pltpu.VMEM((NCHUNK, IDXW), jnp.int32),
            pltpu.VMEM((NCHUNK * NSLOT, IDXW), jnp.int32),
            pltpu.VMEM((CHUNK, 128), jnp.float32),
            pltpu.VMEM((CHUNK, 128), jnp.float32),
            pltpu.SemaphoreType.DMA,
            pltpu.SemaphoreType.DMA,
        ],
        compiler_params=pltpu.CompilerParams(use_tc_tiling_on_sc=False),
    )
    return f(pq, queue2d, champ2d)


MLP_BLK = 4096


def _mlp_body(xrp, rmod, hc, w1r, b1, w2, b2, out):
    rm = rmod[...]
    xp = xrp[...]
    xsel = jnp.where(rm == 0, xp[:, 0:D], 0.0)
    for p in range(1, RPACK):
        xsel = xsel + jnp.where(rm == p, xp[:, p * D:(p + 1) * D], 0.0)
    h = (jnp.dot(xsel, w1r[...], preferred_element_type=jnp.float32)
         + hc[...] + b1[...])
    h = jnp.maximum(h, 0.0)
    o = jnp.dot(h, w2[...], preferred_element_type=jnp.float32) + b2[...]
    out[...] = 1.0 / (1.0 + jnp.exp(-o))


def _mlp(xrp, rmod, hc, W1, b1, W2, b2):
    w1r = W1[:D]
    grid = (B // MLP_BLK,)
    return pl.pallas_call(
        _mlp_body,
        grid=grid,
        in_specs=[
            pl.BlockSpec((MLP_BLK, 128), lambda i: (i, 0)),
            pl.BlockSpec((MLP_BLK, 1), lambda i: (i, 0)),
            pl.BlockSpec((MLP_BLK, 128), lambda i: (i, 0)),
            pl.BlockSpec((D, 128), lambda i: (0, 0)),
            pl.BlockSpec((1, 128), lambda i: (0, 0)),
            pl.BlockSpec((128, 1), lambda i: (0, 0)),
            pl.BlockSpec((1, 1), lambda i: (0, 0)),
        ],
        out_specs=pl.BlockSpec((MLP_BLK, 1), lambda i: (i, 0)),
        out_shape=jax.ShapeDtypeStruct((B, 1), jnp.float32),
    )(xrp, rmod, hc, w1r, b1.reshape(1, 128), W2, b2.reshape(1, 1))


def kernel(emb_region, emb_queue, emb_champ, W1, b1, W2, b2, region, queue_type, champion_ids):
    region = region.astype(jnp.int32)
    queue_type = queue_type.astype(jnp.int32)
    champ_flat = champion_ids.astype(jnp.int32).reshape(B * NSLOT)

    table_t = emb_region.T
    gidx = ((region >> 15) << 13) | (region & (PBG - 1))
    ridx2d = gidx.reshape(B // IDXW, IDXW)
    rmod = ((region >> 13) & 3).reshape(B, 1)
    w1q, w1c = W1[D:2 * D], W1[2 * D:]
    pq = jnp.concatenate(
        [emb_queue @ w1q]
        + [emb_champ @ w1c[D * j:D * (j + 1)] for j in range(NSLOT)], axis=0)
    nrow = NUM_QUEUE + NSLOT * NUM_CHAMP
    REP = 4
    pq_rep = jnp.tile(pq, (REP, 1))

    queue2d = (queue_type + (jnp.arange(B, dtype=jnp.int32) // (B // REP)) * nrow
               ).reshape(B // IDXW, IDXW)
    slot_off = NUM_QUEUE + (jnp.arange(B * NSLOT, dtype=jnp.int32) % NSLOT) * NUM_CHAMP
    rep_off = (jnp.arange(B * NSLOT, dtype=jnp.int32) // (B * NSLOT // REP)) * nrow
    champ2d = (champ_flat + slot_off + rep_off).reshape(B * NSLOT // IDXW, IDXW)

    hc = _sc_small(pq_rep, queue2d, champ2d)
    table128 = _pack(table_t)
    xrp = _sc_region(table128, ridx2d)
    out = _mlp(xrp, rmod, hc, W1, b1, W2, b2)
    return jnp.squeeze(out, axis=1)
